# Initial kernel scaffold; baseline (speedup 1.0000x reference)
import jax
import jax.numpy as jnp
from jax.experimental import pallas as pl


def kernel(x, edge_index, edge_weight, W0, b0, W1, b1):
    raise NotImplementedError("write your pallas kernel here")


# speedup vs baseline: 11.1765x; 11.1765x over previous
import functools

import jax
import jax.numpy as jnp
from jax import lax
from jax.experimental import pallas as pl
from jax.experimental.pallas import tpu as pltpu
from jax.experimental.pallas import tpu_sc as plsc

_NC = 2
_NS = 16
_NW = _NC * _NS
_K = 128
_LANES = 16


def _sc_mesh():
    return plsc.VectorSubcoreMesh(core_axis_name="c", subcore_axis_name="s",
                                  num_cores=_NC, num_subcores=_NS)


def _node_split(n_nodes):
    per = 8 * (-(-n_nodes // (_NS * 8)))
    last = n_nodes - (_NS - 1) * per
    assert 0 < last <= per and last % 8 == 0
    return per, last


def _make_deg(n_nodes, n_edges):
    e_w = n_edges // _NW
    nfull = e_w // _K
    tail = e_w - nfull * _K
    assert e_w % 8 == 0 and tail % _LANES == 0
    per, last = _node_split(n_nodes)

    @functools.partial(
        pl.kernel,
        out_type=jax.ShapeDtypeStruct((_NC * n_nodes,), jnp.float32),
        mesh=_sc_mesh(),
        scratch_types=[
            pltpu.VMEM((_K,), jnp.int32),
            pltpu.VMEM((_K,), jnp.float32),
            pltpu.VMEM((max(tail, _LANES),), jnp.int32),
            pltpu.VMEM((max(tail, _LANES),), jnp.float32),
            pltpu.VMEM((per,), jnp.float32),
            pltpu.VMEM_SHARED((n_nodes,), jnp.float32),
        ],
    )
    def deg_kernel(dst_hbm, ew_hbm, out_hbm,
                   dst_c, ew_c, dst_t, ew_t, zb_v, acc_sh):
        cid = lax.axis_index("c")
        sid = lax.axis_index("s")
        wid = sid * _NC + cid
        base = wid * e_w
        start = sid * per

        def zb_body(i, carry):
            zb_v[pl.ds(i * _LANES, _LANES)] = jnp.zeros((_LANES,), jnp.float32)
            return carry

        lax.fori_loop(0, per // _LANES, zb_body, 0)

        @pl.when(sid < _NS - 1)
        def _():
            pltpu.sync_copy(zb_v, acc_sh.at[pl.ds(start, per)])

        @pl.when(sid == _NS - 1)
        def _():
            pltpu.sync_copy(zb_v.at[pl.ds(0, last)], acc_sh.at[pl.ds(start, last)])

        plsc.subcore_barrier()

        def body(i, carry):
            off = base + i * _K
            pltpu.sync_copy(dst_hbm.at[pl.ds(off, _K)], dst_c)
            pltpu.sync_copy(ew_hbm.at[pl.ds(off, _K)], ew_c)
            pltpu.sync_copy(ew_c, acc_sh.at[dst_c], add=True)
            return carry

        lax.fori_loop(0, nfull, body, 0)
        if tail:
            toff = base + nfull * _K
            pltpu.sync_copy(dst_hbm.at[pl.ds(toff, tail)], dst_t)
            pltpu.sync_copy(ew_hbm.at[pl.ds(toff, tail)], ew_t)
            pltpu.sync_copy(ew_t, acc_sh.at[dst_t], add=True)
        plsc.subcore_barrier()

        @pl.when(sid < _NS - 1)
        def _():
            pltpu.sync_copy(acc_sh.at[pl.ds(start, per)], zb_v)
            pltpu.sync_copy(zb_v, out_hbm.at[pl.ds(cid * n_nodes + start, per)])

        @pl.when(sid == _NS - 1)
        def _():
            pltpu.sync_copy(acc_sh.at[pl.ds(start, last)],
                            zb_v.at[pl.ds(0, last)])
            pltpu.sync_copy(zb_v.at[pl.ds(0, last)],
                            out_hbm.at[pl.ds(cid * n_nodes + start, last)])

    return deg_kernel


def _make_scatter(n_nodes, d, n_edges):
    e_w = n_edges // _NW
    nfull = e_w // _K
    tail = e_w - nfull * _K
    assert e_w % 8 == 0 and tail % _LANES == 0
    per, last = _node_split(n_nodes)
    tl = max(tail, _LANES)

    def _scale_rows(rows_ref, w_ref, count):
        def scale(g, c2):
            w16 = w_ref[pl.ds(g * _LANES, _LANES)]
            for l in range(_LANES):
                w = w16[l]
                k = g * _LANES + l
                for j in range(d // _LANES):
                    sl = pl.ds(j * _LANES, _LANES)
                    rows_ref[k, sl] = rows_ref[k, sl] * w
            return c2

        lax.fori_loop(0, count // _LANES, scale, 0)

    @functools.partial(
        pl.kernel,
        out_type=jax.ShapeDtypeStruct((_NC, n_nodes, d), jnp.float32),
        mesh=_sc_mesh(),
        scratch_types=[
            pltpu.VMEM((_K,), jnp.int32),
            pltpu.VMEM((_K,), jnp.int32),
            pltpu.VMEM((_K,), jnp.float32),
            pltpu.VMEM((tl,), jnp.int32),
            pltpu.VMEM((tl,), jnp.int32),
            pltpu.VMEM((tl,), jnp.float32),
            pltpu.VMEM((_K, d), jnp.float32),
            pltpu.VMEM((tl, d), jnp.float32),
            pltpu.VMEM_SHARED((n_nodes, d), jnp.float32),
            pltpu.SemaphoreType.DMA,
        ],
    )
    def scat_kernel(h_hbm, src_hbm, dst_hbm, ew_hbm, out_hbm,
                    src_c, dst_c, ew_c, src_t, dst_t, ew_t,
                    rows_v, rows_t, acc_sh, gsem):
        cid = lax.axis_index("c")
        sid = lax.axis_index("s")
        wid = sid * _NC + cid
        base = wid * e_w
        start = sid * per

        def zb_body(i, carry):
            for j in range(d // _LANES):
                rows_v[i, pl.ds(j * _LANES, _LANES)] = jnp.zeros(
                    (_LANES,), jnp.float32)
            return carry

        lax.fori_loop(0, _K, zb_body, 0)

        nz_full = per // _K
        ztail = per - nz_full * _K
        ztail_last = last - (last // _K) * _K

        def zfill(i, carry):
            pltpu.sync_copy(rows_v, acc_sh.at[pl.ds(start + i * _K, _K)])
            return carry

        nz = jnp.where(sid == _NS - 1, last // _K, nz_full)
        lax.fori_loop(0, nz, zfill, 0)
        if ztail:
            @pl.when(sid < _NS - 1)
            def _():
                pltpu.sync_copy(
                    rows_v.at[pl.ds(0, ztail)],
                    acc_sh.at[pl.ds(start + nz_full * _K, ztail)])
        if ztail_last:
            @pl.when(sid == _NS - 1)
            def _():
                pltpu.sync_copy(
                    rows_v.at[pl.ds(0, ztail_last)],
                    acc_sh.at[pl.ds(start + (last // _K) * _K, ztail_last)])
        plsc.subcore_barrier()

        def chunk(i, carry):
            off = base + i * _K
            pltpu.sync_copy(src_hbm.at[pl.ds(off, _K)], src_c)
            pltpu.sync_copy(dst_hbm.at[pl.ds(off, _K)], dst_c)
            pltpu.sync_copy(ew_hbm.at[pl.ds(off, _K)], ew_c)
            pltpu.async_copy(h_hbm.at[src_c], rows_v, gsem).wait()
            _scale_rows(rows_v, ew_c, _K)
            pltpu.sync_copy(rows_v, acc_sh.at[dst_c], add=True)
            return carry

        lax.fori_loop(0, nfull, chunk, 0)
        if tail:
            toff = base + nfull * _K
            pltpu.sync_copy(src_hbm.at[pl.ds(toff, tail)], src_t)
            pltpu.sync_copy(dst_hbm.at[pl.ds(toff, tail)], dst_t)
            pltpu.sync_copy(ew_hbm.at[pl.ds(toff, tail)], ew_t)
            pltpu.async_copy(h_hbm.at[src_t], rows_t, gsem).wait()
            _scale_rows(rows_t, ew_t, tail)
            pltpu.sync_copy(rows_t, acc_sh.at[dst_t], add=True)
        plsc.subcore_barrier()

        def cout(i, carry):
            r = start + i * _K
            pltpu.sync_copy(acc_sh.at[pl.ds(r, _K)], rows_v)
            pltpu.sync_copy(rows_v, out_hbm.at[cid, pl.ds(r, _K)])
            return carry

        lax.fori_loop(0, nz, cout, 0)
        if ztail:
            @pl.when(sid < _NS - 1)
            def _():
                r = start + nz_full * _K
                pltpu.sync_copy(acc_sh.at[pl.ds(r, ztail)],
                                rows_v.at[pl.ds(0, ztail)])
                pltpu.sync_copy(rows_v.at[pl.ds(0, ztail)],
                                out_hbm.at[cid, pl.ds(r, ztail)])
        if ztail_last:
            @pl.when(sid == _NS - 1)
            def _():
                r = start + (last // _K) * _K
                pltpu.sync_copy(acc_sh.at[pl.ds(r, ztail_last)],
                                rows_v.at[pl.ds(0, ztail_last)])
                pltpu.sync_copy(rows_v.at[pl.ds(0, ztail_last)],
                                out_hbm.at[cid, pl.ds(r, ztail_last)])

    return scat_kernel


def _mm_scale_body(deg_ref, x_ref, w_ref, h_ref, dinv_ref):
    deg = deg_ref[0] + deg_ref[1] + 1.0
    dinv = jnp.where(deg > 0, lax.rsqrt(deg), 0.0)
    dinv_ref[...] = dinv
    h = jnp.dot(x_ref[...], w_ref[...], preferred_element_type=jnp.float32)
    h_ref[...] = h * dinv


def _combine_mm_body(a_ref, h_ref, dinv_ref, b_ref, w_ref, f_ref, h2_ref):
    dinv = dinv_ref[...]
    t = (a_ref[0] + a_ref[1] + h_ref[...]) * dinv + b_ref[...]
    t = jnp.maximum(t, 0.0)
    nrm = jnp.sqrt(jnp.sum(t * t, axis=1, keepdims=True))
    f = t / jnp.maximum(nrm, 1e-12)
    f_ref[...] = f
    h2 = jnp.dot(f, w_ref[...], preferred_element_type=jnp.float32)
    h2_ref[...] = h2 * dinv


def _combine_final_body(a_ref, h_ref, dinv_ref, b_ref, f_ref):
    t = (a_ref[0] + a_ref[1] + h_ref[...]) * dinv_ref[...] + b_ref[...]
    t = jnp.maximum(t, 0.0)
    nrm = jnp.sqrt(jnp.sum(t * t, axis=1, keepdims=True))
    f_ref[...] = t / jnp.maximum(nrm, 1e-12)


def _row_grid(n_nodes):
    r = 1000 if n_nodes % 1000 == 0 else n_nodes
    return r, n_nodes // r


def _mm_scale(deg3, x, w):
    n, d = x.shape
    r, g = _row_grid(n)
    return pl.pallas_call(
        _mm_scale_body,
        grid=(g,),
        in_specs=[
            pl.BlockSpec((_NC, r, 1), lambda i: (0, i, 0)),
            pl.BlockSpec((r, d), lambda i: (i, 0)),
            pl.BlockSpec((d, d), lambda i: (0, 0)),
        ],
        out_specs=[
            pl.BlockSpec((r, d), lambda i: (i, 0)),
            pl.BlockSpec((r, 1), lambda i: (i, 0)),
        ],
        out_shape=[
            jax.ShapeDtypeStruct((n, d), jnp.float32),
            jax.ShapeDtypeStruct((n, 1), jnp.float32),
        ],
    )(deg3, x, w)


def _combine_mm(acc, h, dinv, b2, w):
    n, d = h.shape
    r, g = _row_grid(n)
    return pl.pallas_call(
        _combine_mm_body,
        grid=(g,),
        in_specs=[
            pl.BlockSpec((_NC, r, d), lambda i: (0, i, 0)),
            pl.BlockSpec((r, d), lambda i: (i, 0)),
            pl.BlockSpec((r, 1), lambda i: (i, 0)),
            pl.BlockSpec((1, d), lambda i: (0, 0)),
            pl.BlockSpec((d, d), lambda i: (0, 0)),
        ],
        out_specs=[
            pl.BlockSpec((r, d), lambda i: (i, 0)),
            pl.BlockSpec((r, d), lambda i: (i, 0)),
        ],
        out_shape=[
            jax.ShapeDtypeStruct((n, d), jnp.float32),
            jax.ShapeDtypeStruct((n, d), jnp.float32),
        ],
    )(acc, h, dinv, b2, w)


def _combine_final(acc, h, dinv, b2):
    n, d = h.shape
    r, g = _row_grid(n)
    return pl.pallas_call(
        _combine_final_body,
        grid=(g,),
        in_specs=[
            pl.BlockSpec((_NC, r, d), lambda i: (0, i, 0)),
            pl.BlockSpec((r, d), lambda i: (i, 0)),
            pl.BlockSpec((r, 1), lambda i: (i, 0)),
            pl.BlockSpec((1, d), lambda i: (0, 0)),
        ],
        out_specs=pl.BlockSpec((r, d), lambda i: (i, 0)),
        out_shape=jax.ShapeDtypeStruct((n, d), jnp.float32),
    )(acc, h, dinv, b2)


def kernel(x, edge_index, edge_weight, W0, b0, W1, b1):
    n, d = x.shape
    e = edge_index.shape[1]
    src = edge_index[0]
    dst = edge_index[1]
    ew = edge_weight

    deg_p = _make_deg(n, e)(dst, ew)
    deg3 = deg_p.reshape(_NC, n, 1)
    h0p, dinv = _mm_scale(deg3, x, W0)

    scat = _make_scatter(n, d, e)
    acc0 = scat(h0p, src, dst, ew)
    f0, h1p = _combine_mm(acc0, h0p, dinv, b0.reshape(1, d), W1)
    acc1 = scat(h1p, src, dst, ew)
    f1 = _combine_final(acc1, h1p, dinv, b1.reshape(1, d))
    return (x, f0, f1)


# speedup vs baseline: 23.3759x; 2.0915x over previous
import functools

import jax
import jax.numpy as jnp
from jax import lax
from jax.experimental import pallas as pl
from jax.experimental.pallas import tpu as pltpu
from jax.experimental.pallas import tpu_sc as plsc

_NC = 2
_NS = 16
_NW = _NC * _NS
_K = 128
_LANES = 16


def _sc_mesh():
    return plsc.VectorSubcoreMesh(core_axis_name="c", subcore_axis_name="s",
                                  num_cores=_NC, num_subcores=_NS)


def _node_split(n_nodes):
    per = 8 * (-(-n_nodes // (_NS * 8)))
    last = n_nodes - (_NS - 1) * per
    assert 0 < last <= per and last % 8 == 0
    return per, last


def _vec_copy(src_ref, src_off, dst_ref, n):
    for g in range(n // _LANES):
        dst_ref[pl.ds(g * _LANES, _LANES)] = (
            src_ref[pl.ds(src_off + g * _LANES, _LANES)])


def _make_deg(n_nodes, n_edges):
    e_w = n_edges // _NW
    nfull = e_w // _K
    tail = e_w - nfull * _K
    assert e_w % 8 == 0 and tail % _LANES == 0 and nfull % 2 == 0
    per, last = _node_split(n_nodes)
    tl = max(tail, _LANES)
    zb_n = _LANES * (-(-per // _LANES))

    @functools.partial(
        pl.kernel,
        out_type=jax.ShapeDtypeStruct((_NC * n_nodes,), jnp.float32),
        mesh=_sc_mesh(),
        scratch_types=[
            pltpu.VMEM((e_w,), jnp.float32),
            pltpu.VMEM((_K,), jnp.int32),
            pltpu.VMEM((_K,), jnp.int32),
            pltpu.VMEM((tl,), jnp.int32),
            pltpu.VMEM((zb_n,), jnp.float32),
            pltpu.VMEM_SHARED((n_nodes,), jnp.float32),
            pltpu.SemaphoreType.DMA,
            pltpu.SemaphoreType.DMA,
            pltpu.SemaphoreType.DMA,
            pltpu.SemaphoreType.DMA,
            pltpu.SemaphoreType.DMA,
        ],
    )
    def deg_kernel(dst_hbm, ew_hbm, out_hbm,
                   ew_all, dc0, dc1, dst_t, zb_v, acc_sh,
                   tsem, s0, s1, d0, d1):
        cid = lax.axis_index("c")
        sid = lax.axis_index("s")
        wid = sid * _NC + cid
        base = wid * e_w
        start = sid * per
        dstc = (dc0, dc1)
        ssem = (s0, s1)
        dsem = (d0, d1)

        st2 = pltpu.async_copy(ew_hbm.at[pl.ds(base, e_w)], ew_all, tsem)
        pltpu.async_copy(dst_hbm.at[pl.ds(base, _K)], dc0, d0)

        def zb_body(i, carry):
            zb_v[pl.ds(i * _LANES, _LANES)] = jnp.zeros((_LANES,), jnp.float32)
            return carry

        lax.fori_loop(0, zb_n // _LANES, zb_body, 0)

        @pl.when(sid < _NS - 1)
        def _():
            pltpu.sync_copy(zb_v.at[pl.ds(0, per)], acc_sh.at[pl.ds(start, per)])

        @pl.when(sid == _NS - 1)
        def _():
            pltpu.sync_copy(zb_v.at[pl.ds(0, last)], acc_sh.at[pl.ds(start, last)])

        st2.wait()
        plsc.subcore_barrier()

        def body(t, carry):
            for b in range(2):
                i = t * 2 + b
                p = b
                q = 1 - b
                pltpu.make_async_copy(
                    dst_hbm.at[pl.ds(base, _K)], dstc[p], dsem[p]).wait()

                @pl.when(i >= 1)
                def _():
                    pltpu.make_async_copy(
                        ew_all.at[pl.ds(0, _K)], acc_sh.at[dstc[q]],
                        ssem[q]).wait()

                @pl.when(i + 1 < nfull)
                def _():
                    pltpu.async_copy(
                        dst_hbm.at[pl.ds(base + (i + 1) * _K, _K)],
                        dstc[q], dsem[q])

                pltpu.async_copy(ew_all.at[pl.ds(i * _K, _K)],
                                 acc_sh.at[dstc[p]], ssem[p], add=True)
            return carry

        lax.fori_loop(0, nfull // 2, body, 0)
        if nfull >= 1:
            lp = (nfull - 1) % 2
            pltpu.make_async_copy(ew_all.at[pl.ds(0, _K)],
                                  acc_sh.at[dstc[lp]], ssem[lp]).wait()
        if tail:
            toff = nfull * _K
            pltpu.sync_copy(dst_hbm.at[pl.ds(base + toff, tail)], dst_t)
            pltpu.sync_copy(ew_all.at[pl.ds(toff, tail)],
                            acc_sh.at[dst_t], add=True)
        plsc.subcore_barrier()

        @pl.when(sid < _NS - 1)
        def _():
            pltpu.sync_copy(acc_sh.at[pl.ds(start, per)],
                            zb_v.at[pl.ds(0, per)])
            pltpu.sync_copy(zb_v.at[pl.ds(0, per)],
                            out_hbm.at[pl.ds(cid * n_nodes + start, per)])

        @pl.when(sid == _NS - 1)
        def _():
            pltpu.sync_copy(acc_sh.at[pl.ds(start, last)],
                            zb_v.at[pl.ds(0, last)])
            pltpu.sync_copy(zb_v.at[pl.ds(0, last)],
                            out_hbm.at[pl.ds(cid * n_nodes + start, last)])

    return deg_kernel


def _make_scatter(n_nodes, d, n_edges):
    e_w = n_edges // _NW
    nfull = e_w // _K
    tail = e_w - nfull * _K
    assert e_w % 8 == 0 and tail % _LANES == 0 and nfull % 2 == 0
    per, last = _node_split(n_nodes)
    tl = max(tail, _LANES)

    def _scale_rows(rows_ref, w_ref, w_off, count):
        def scale(g, c2):
            w16 = w_ref[pl.ds(w_off + g * _LANES, _LANES)]
            for l in range(_LANES):
                w = w16[l]
                k = g * _LANES + l
                for j in range(d // _LANES):
                    sl = pl.ds(j * _LANES, _LANES)
                    rows_ref[k, sl] = rows_ref[k, sl] * w
            return c2

        lax.fori_loop(0, count // _LANES, scale, 0)

    @functools.partial(
        pl.kernel,
        out_type=jax.ShapeDtypeStruct((_NC, n_nodes, d), jnp.float32),
        mesh=_sc_mesh(),
        scratch_types=[
            pltpu.VMEM((e_w,), jnp.int32),
            pltpu.VMEM((_K, d), jnp.float32),
            pltpu.VMEM((_K, d), jnp.float32),
            pltpu.VMEM((_K,), jnp.int32),
            pltpu.VMEM((_K,), jnp.int32),
            pltpu.VMEM((_K,), jnp.float32),
            pltpu.VMEM((_K,), jnp.float32),
            pltpu.VMEM((tl, d), jnp.float32),
            pltpu.VMEM((tl,), jnp.int32),
            pltpu.VMEM_SHARED((n_nodes, d), jnp.float32),
            pltpu.SemaphoreType.DMA,
            pltpu.SemaphoreType.DMA,
            pltpu.SemaphoreType.DMA,
            pltpu.SemaphoreType.DMA,
            pltpu.SemaphoreType.DMA,
            pltpu.SemaphoreType.DMA,
            pltpu.SemaphoreType.DMA,
        ],
    )
    def scat_kernel(h_hbm, src_hbm, dst_hbm, ew_hbm, out_hbm,
                    src_all, rows0, rows1, dc0, dc1, ewc0, ewc1,
                    rows_t, dst_t, acc_sh, tsem, g0, g1, s0, s1, d0, d1):
        cid = lax.axis_index("c")
        sid = lax.axis_index("s")
        wid = sid * _NC + cid
        base = wid * e_w
        start = sid * per
        rows = (rows0, rows1)
        dstc = (dc0, dc1)
        ewc = (ewc0, ewc1)
        gsem = (g0, g1)
        ssem = (s0, s1)
        dsem = (d0, d1)

        st1 = pltpu.async_copy(src_hbm.at[pl.ds(base, e_w)], src_all, tsem)
        pltpu.async_copy(dst_hbm.at[pl.ds(base, _K)], dc0, d0)
        pltpu.async_copy(ew_hbm.at[pl.ds(base, _K)], ewc0, d0)

        def zb_body(i, carry):
            for j in range(d // _LANES):
                rows0[i, pl.ds(j * _LANES, _LANES)] = jnp.zeros(
                    (_LANES,), jnp.float32)
            return carry

        lax.fori_loop(0, _K, zb_body, 0)

        nz_full = per // _K
        ztail = per - nz_full * _K
        ztail_last = last - (last // _K) * _K

        def zfill(i, carry):
            pltpu.sync_copy(rows0, acc_sh.at[pl.ds(start + i * _K, _K)])
            return carry

        nz = jnp.where(sid == _NS - 1, last // _K, nz_full)
        lax.fori_loop(0, nz, zfill, 0)
        if ztail:
            @pl.when(sid < _NS - 1)
            def _():
                pltpu.sync_copy(
                    rows0.at[pl.ds(0, ztail)],
                    acc_sh.at[pl.ds(start + nz_full * _K, ztail)])
        if ztail_last:
            @pl.when(sid == _NS - 1)
            def _():
                pltpu.sync_copy(
                    rows0.at[pl.ds(0, ztail_last)],
                    acc_sh.at[pl.ds(start + (last // _K) * _K, ztail_last)])
        st1.wait()
        plsc.subcore_barrier()

        pltpu.async_copy(h_hbm.at[src_all.at[pl.ds(0, _K)]], rows0, g0)

        def super_chunk(t, carry):
            for b in range(2):
                i = t * 2 + b
                p = b
                q = 1 - b

                pltpu.make_async_copy(
                    h_hbm.at[src_all.at[pl.ds(i * _K, _K)]],
                    rows[p], gsem[p]).wait()
                pltpu.make_async_copy(
                    dst_hbm.at[pl.ds(base, _K)], dstc[p], dsem[p]).wait()
                pltpu.make_async_copy(
                    ew_hbm.at[pl.ds(base, _K)], ewc[p], dsem[p]).wait()

                @pl.when(i >= 1)
                def _():
                    pltpu.make_async_copy(
                        rows[q], acc_sh.at[dstc[q]], ssem[q]).wait()

                @pl.when(i + 1 < nfull)
                def _():
                    off = base + (i + 1) * _K
                    pltpu.async_copy(dst_hbm.at[pl.ds(off, _K)], dstc[q],
                                     dsem[q])
                    pltpu.async_copy(ew_hbm.at[pl.ds(off, _K)], ewc[q],
                                     dsem[q])
                    pltpu.async_copy(
                        h_hbm.at[src_all.at[pl.ds((i + 1) * _K, _K)]],
                        rows[q], gsem[q])

                _scale_rows(rows[p], ewc[p], 0, _K)
                pltpu.async_copy(rows[p], acc_sh.at[dstc[p]], ssem[p],
                                 add=True)
            return carry

        lax.fori_loop(0, nfull // 2, super_chunk, 0)
        if nfull >= 1:
            lp = (nfull - 1) % 2
            pltpu.make_async_copy(rows[lp], acc_sh.at[dstc[lp]],
                                  ssem[lp]).wait()
        if tail:
            toff = nfull * _K
            pltpu.sync_copy(dst_hbm.at[pl.ds(base + toff, tail)], dst_t)
            pltpu.sync_copy(ew_hbm.at[pl.ds(base + toff, tail)],
                            ewc0.at[pl.ds(0, tail)])
            pltpu.async_copy(
                h_hbm.at[src_all.at[pl.ds(toff, tail)]], rows_t, tsem).wait()
            _scale_rows(rows_t, ewc0, 0, tail)
            pltpu.sync_copy(rows_t, acc_sh.at[dst_t], add=True)
        plsc.subcore_barrier()

        def cout(i, carry):
            r = start + i * _K
            pltpu.sync_copy(acc_sh.at[pl.ds(r, _K)], rows0)
            pltpu.sync_copy(rows0, out_hbm.at[cid, pl.ds(r, _K)])
            return carry

        lax.fori_loop(0, nz, cout, 0)
        if ztail:
            @pl.when(sid < _NS - 1)
            def _():
                r = start + nz_full * _K
                pltpu.sync_copy(acc_sh.at[pl.ds(r, ztail)],
                                rows0.at[pl.ds(0, ztail)])
                pltpu.sync_copy(rows0.at[pl.ds(0, ztail)],
                                out_hbm.at[cid, pl.ds(r, ztail)])
        if ztail_last:
            @pl.when(sid == _NS - 1)
            def _():
                r = start + (last // _K) * _K
                pltpu.sync_copy(acc_sh.at[pl.ds(r, ztail_last)],
                                rows0.at[pl.ds(0, ztail_last)])
                pltpu.sync_copy(rows0.at[pl.ds(0, ztail_last)],
                                out_hbm.at[cid, pl.ds(r, ztail_last)])

    return scat_kernel


def _mm_scale_body(deg_ref, x_ref, w_ref, h_ref, dinv_ref):
    deg = deg_ref[0] + deg_ref[1] + 1.0
    dinv = jnp.where(deg > 0, lax.rsqrt(deg), 0.0)
    dinv_ref[...] = dinv
    h = jnp.dot(x_ref[...], w_ref[...], preferred_element_type=jnp.float32)
    h_ref[...] = h * dinv


def _combine_mm_body(a_ref, h_ref, dinv_ref, b_ref, w_ref, f_ref, h2_ref):
    dinv = dinv_ref[...]
    t = (a_ref[0] + a_ref[1] + h_ref[...]) * dinv + b_ref[...]
    t = jnp.maximum(t, 0.0)
    nrm = jnp.sqrt(jnp.sum(t * t, axis=1, keepdims=True))
    f = t / jnp.maximum(nrm, 1e-12)
    f_ref[...] = f
    h2 = jnp.dot(f, w_ref[...], preferred_element_type=jnp.float32)
    h2_ref[...] = h2 * dinv


def _combine_final_body(a_ref, h_ref, dinv_ref, b_ref, f_ref):
    t = (a_ref[0] + a_ref[1] + h_ref[...]) * dinv_ref[...] + b_ref[...]
    t = jnp.maximum(t, 0.0)
    nrm = jnp.sqrt(jnp.sum(t * t, axis=1, keepdims=True))
    f_ref[...] = t / jnp.maximum(nrm, 1e-12)


def _row_grid(n_nodes):
    r = 1000 if n_nodes % 1000 == 0 else n_nodes
    return r, n_nodes // r


def _mm_scale(deg3, x, w):
    n, d = x.shape
    r, g = _row_grid(n)
    return pl.pallas_call(
        _mm_scale_body,
        grid=(g,),
        in_specs=[
            pl.BlockSpec((_NC, r, 1), lambda i: (0, i, 0)),
            pl.BlockSpec((r, d), lambda i: (i, 0)),
            pl.BlockSpec((d, d), lambda i: (0, 0)),
        ],
        out_specs=[
            pl.BlockSpec((r, d), lambda i: (i, 0)),
            pl.BlockSpec((r, 1), lambda i: (i, 0)),
        ],
        out_shape=[
            jax.ShapeDtypeStruct((n, d), jnp.float32),
            jax.ShapeDtypeStruct((n, 1), jnp.float32),
        ],
    )(deg3, x, w)


def _combine_mm(acc, h, dinv, b2, w):
    n, d = h.shape
    r, g = _row_grid(n)
    return pl.pallas_call(
        _combine_mm_body,
        grid=(g,),
        in_specs=[
            pl.BlockSpec((_NC, r, d), lambda i: (0, i, 0)),
            pl.BlockSpec((r, d), lambda i: (i, 0)),
            pl.BlockSpec((r, 1), lambda i: (i, 0)),
            pl.BlockSpec((1, d), lambda i: (0, 0)),
            pl.BlockSpec((d, d), lambda i: (0, 0)),
        ],
        out_specs=[
            pl.BlockSpec((r, d), lambda i: (i, 0)),
            pl.BlockSpec((r, d), lambda i: (i, 0)),
        ],
        out_shape=[
            jax.ShapeDtypeStruct((n, d), jnp.float32),
            jax.ShapeDtypeStruct((n, d), jnp.float32),
        ],
    )(acc, h, dinv, b2, w)


def _combine_final(acc, h, dinv, b2):
    n, d = h.shape
    r, g = _row_grid(n)
    return pl.pallas_call(
        _combine_final_body,
        grid=(g,),
        in_specs=[
            pl.BlockSpec((_NC, r, d), lambda i: (0, i, 0)),
            pl.BlockSpec((r, d), lambda i: (i, 0)),
            pl.BlockSpec((r, 1), lambda i: (i, 0)),
            pl.BlockSpec((1, d), lambda i: (0, 0)),
        ],
        out_specs=pl.BlockSpec((r, d), lambda i: (i, 0)),
        out_shape=jax.ShapeDtypeStruct((n, d), jnp.float32),
    )(acc, h, dinv, b2)


def kernel(x, edge_index, edge_weight, W0, b0, W1, b1):
    n, d = x.shape
    e = edge_index.shape[1]
    src = edge_index[0]
    dst = edge_index[1]
    ew = edge_weight

    deg_p = _make_deg(n, e)(dst, ew)
    deg3 = deg_p.reshape(_NC, n, 1)
    h0p, dinv = _mm_scale(deg3, x, W0)

    scat = _make_scatter(n, d, e)
    acc0 = scat(h0p, src, dst, ew)
    f0, h1p = _combine_mm(acc0, h0p, dinv, b0.reshape(1, d), W1)
    acc1 = scat(h1p, src, dst, ew)
    f1 = _combine_final(acc1, h1p, dinv, b1.reshape(1, d))
    return (x, f0, f1)
